# baseline (speedup 1.0000x reference)
import functools

import jax
import jax.numpy as jnp
from jax import lax
from jax.experimental import pallas as pl
from jax.experimental.pallas import tpu as pltpu
from jax.experimental.pallas import tpu_sc as plsc

T_STEPS = 4
_LANES = 128


def _round_up(x, m):
    return ((x + m - 1) // m) * m


def _tanh(x):
    x = jnp.clip(x, -7.99881172180175781, 7.99881172180175781)
    x2 = x * x
    p = x2 * -2.76076847742355e-16 + 2.00018790482477e-13
    p = p * x2 + -8.60467152213735e-11
    p = p * x2 + 5.12229709037114e-08
    p = p * x2 + 1.48572235717979e-05
    p = p * x2 + 6.37261928875436e-04
    p = p * x2 + 4.89352455891786e-03
    q = x2 * 1.19825839466702e-06 + 1.18534705686654e-04
    q = q * x2 + 2.26843463243900e-03
    q = q * x2 + 4.89352518554385e-03
    inv = 1.0 / q
    inv = inv * (2.0 - q * inv)
    return x * p * inv


def _sigmoid(x):
    return 0.5 + 0.5 * _tanh(0.5 * x)


def _sc_gather(table, idx3d, n_rows):
    info = plsc.get_sparse_core_info()
    nw = info.num_cores * info.num_subcores
    per = n_rows // nw
    n128 = per // 128
    sub = 1
    chunks = n128 // sub
    nbuf = 3
    mesh = plsc.VectorSubcoreMesh(core_axis_name="c", subcore_axis_name="s")

    n_table = table.shape[0]
    rps = n_table // info.num_subcores

    @functools.partial(
        pl.kernel,
        mesh=mesh,
        out_type=jax.ShapeDtypeStruct((n_rows, _LANES), jnp.float32),
        scratch_types=[
            pltpu.VMEM((n128, 128), jnp.int32),
            pltpu.VMEM((128, _LANES), jnp.float32),
            pltpu.VMEM((128, _LANES), jnp.float32),
            pltpu.VMEM((128, _LANES), jnp.float32),
            pltpu.VMEM_SHARED((n_table, _LANES), jnp.float32),
            pltpu.SemaphoreType.DMA,
            pltpu.SemaphoreType.DMA,
            pltpu.SemaphoreType.DMA,
        ],
    )
    def k(table_hbm, idx_hbm, out_hbm, idx_v, b0, b1, b2, table_sh, s0, s1, s2):
        bufs = (b0, b1, b2)
        sems = (s0, s1, s2)
        cid = lax.axis_index("c")
        sid = lax.axis_index("s")
        wid = sid * info.num_cores + cid
        pltpu.sync_copy(table_hbm.at[pl.ds(sid * rps, rps)],
                        table_sh.at[pl.ds(sid * rps, rps)])
        pltpu.sync_copy(idx_hbm.at[wid], idx_v)
        plsc.subcore_barrier()
        base = wid * per

        def start(j):
            return pltpu.async_copy(
                table_sh.at[idx_v.at[j]],
                bufs[j % nbuf], sems[j % nbuf])

        handles = {}
        for j in range(min(nbuf, chunks)):
            handles[j] = start(j)
        for j in range(chunks):
            handles.pop(j).wait()
            pltpu.sync_copy(bufs[j % nbuf],
                            out_hbm.at[pl.ds(base + j * 128, 128)])
            if j + nbuf < chunks:
                handles[j + nbuf] = start(j + nbuf)

    return k(table, idx3d)


def _sc_scatter(h, idx3d, zeros_nl, nl_pad, p_pad, seq_len):
    info = plsc.get_sparse_core_info()
    nc, ns = info.num_cores, info.num_subcores
    nw = nc * ns
    n_rows = seq_len * p_pad
    per = n_rows // nw
    chunks = per // 128
    half = p_pad // nc
    rps = nl_pad // ns
    mesh = plsc.VectorSubcoreMesh(core_axis_name="c", subcore_axis_name="s")

    @functools.partial(
        pl.kernel,
        mesh=mesh,
        out_type=jax.ShapeDtypeStruct((nc, nl_pad, _LANES), jnp.float32),
        scratch_types=[
            pltpu.VMEM((chunks, 128), jnp.int32),
            pltpu.VMEM((128, _LANES), jnp.float32),
            pltpu.VMEM((128, _LANES), jnp.float32),
            pltpu.VMEM_SHARED((nl_pad, _LANES), jnp.float32),
            pltpu.VMEM_SHARED((p_pad // 2, _LANES), jnp.float32),
            pltpu.SemaphoreType.DMA,
            pltpu.SemaphoreType.DMA,
        ],
    )
    def k(h_hbm, idx_hbm, z_hbm, out_hbm, idx_v, r0, r1, accum, h_sh, s0, s1):
        bufs = (r0, r1)
        sems = (s0, s1)
        cid = lax.axis_index("c")
        sid = lax.axis_index("s")
        wid = sid * nc + cid
        hps = half // ns
        pltpu.sync_copy(z_hbm.at[pl.ds(sid * rps, rps)],
                        accum.at[pl.ds(sid * rps, rps)])
        pltpu.sync_copy(h_hbm.at[pl.ds(cid * half + sid * hps, hps)],
                        h_sh.at[pl.ds(sid * hps, hps)])
        pltpu.sync_copy(idx_hbm.at[wid], idx_v)
        plsc.subcore_barrier()

        def start(j):
            return pltpu.async_copy(
                h_sh.at[pl.ds(j * 128, 128)], bufs[j % 2], sems[j % 2])

        handles = {}
        for j in range(min(2, chunks)):
            handles[j] = start(j)
        for j in range(chunks):
            handles.pop(j).wait()
            pltpu.sync_copy(bufs[j % 2], accum.at[idx_v.at[j]], add=True)
            if j + 2 < chunks:
                handles[j + 2] = start(j + 2)
        plsc.subcore_barrier()
        pltpu.sync_copy(accum.at[pl.ds(sid * rps, rps)],
                        out_hbm.at[cid, pl.ds(sid * rps, rps)])

    return k(h, idx3d, zeros_nl)


def _path_gru(x3, h, wih_t, whh_t, bih, bhh, seq_len, p_pad, hp):
    BP = 1024
    g3 = 3 * hp

    def body(x_ref, h_ref, wih_ref, whh_ref, bih_ref, bhh_ref, out_ref):
        lens = jnp.zeros((BP,), jnp.int32)
        for t in range(seq_len):
            lens = lens + jnp.any(x_ref[t] != 0.0, axis=1).astype(jnp.int32)
        h_cur = h_ref[...]
        wih = wih_ref[...]
        whh = whh_ref[...]
        bih = bih_ref[...]
        bhh = bhh_ref[...]
        for t in range(seq_len):
            xt = x_ref[t]
            gi = jnp.dot(xt, wih, preferred_element_type=jnp.float32) + bih
            gh = jnp.dot(h_cur, whh, preferred_element_type=jnp.float32) + bhh
            r = jax.nn.sigmoid(gi[:, :hp] + gh[:, :hp])
            z = jax.nn.sigmoid(gi[:, hp:2 * hp] + gh[:, hp:2 * hp])
            n = jnp.tanh(gi[:, 2 * hp:] + r * gh[:, 2 * hp:])
            hn = (1.0 - z) * n + z * h_cur
            h_cur = jnp.where((t < lens)[:, None], hn, h_cur)
        out_ref[...] = h_cur

    return pl.pallas_call(
        body,
        grid=(p_pad // BP,),
        in_specs=[
            pl.BlockSpec((seq_len, BP, _LANES), lambda i: (0, i, 0)),
            pl.BlockSpec((BP, _LANES), lambda i: (i, 0)),
            pl.BlockSpec((_LANES, g3), lambda i: (0, 0)),
            pl.BlockSpec((_LANES, g3), lambda i: (0, 0)),
            pl.BlockSpec((1, g3), lambda i: (0, 0)),
            pl.BlockSpec((1, g3), lambda i: (0, 0)),
        ],
        out_specs=pl.BlockSpec((BP, _LANES), lambda i: (i, 0)),
        out_shape=jax.ShapeDtypeStruct((p_pad, _LANES), jnp.float32),
    )(x3, h, wih_t, whh_t, bih, bhh)


def _link_update(part0, part1, link, cwih_t, cwhh_t, cbih, cbhh, nl_pad, hl):
    BL = 512
    g3 = 3 * hl

    def body(p0_ref, p1_ref, ls_ref, wih_ref, whh_ref, bih_ref, bhh_ref, out_ref):
        ps = p0_ref[...] + p1_ref[...]
        ls = ls_ref[...]
        gi = jnp.dot(ps, wih_ref[...], preferred_element_type=jnp.float32) + bih_ref[...]
        gh = jnp.dot(ls, whh_ref[...], preferred_element_type=jnp.float32) + bhh_ref[...]
        r = jax.nn.sigmoid(gi[:, :hl] + gh[:, :hl])
        z = jax.nn.sigmoid(gi[:, hl:2 * hl] + gh[:, hl:2 * hl])
        n = jnp.tanh(gi[:, 2 * hl:] + r * gh[:, 2 * hl:])
        out_ref[...] = (1.0 - z) * n + z * ls

    return pl.pallas_call(
        body,
        grid=(nl_pad // BL,),
        in_specs=[
            pl.BlockSpec((BL, _LANES), lambda i: (i, 0)),
            pl.BlockSpec((BL, _LANES), lambda i: (i, 0)),
            pl.BlockSpec((BL, _LANES), lambda i: (i, 0)),
            pl.BlockSpec((_LANES, g3), lambda i: (0, 0)),
            pl.BlockSpec((_LANES, g3), lambda i: (0, 0)),
            pl.BlockSpec((1, g3), lambda i: (0, 0)),
            pl.BlockSpec((1, g3), lambda i: (0, 0)),
        ],
        out_specs=pl.BlockSpec((BL, _LANES), lambda i: (i, 0)),
        out_shape=jax.ShapeDtypeStruct((nl_pad, _LANES), jnp.float32),
    )(part0, part1, link, cwih_t, cwhh_t, cbih, cbhh)


def _readout(h, w1_t, b1, w2_t, b2, w3, b3, p_pad, ru):
    BP = 1024

    def body(h_ref, w1_ref, b1_ref, w2_ref, b2_ref, w3_ref, b3_ref, out_ref):
        r1 = jnp.maximum(
            jnp.dot(h_ref[...], w1_ref[...], preferred_element_type=jnp.float32)
            + b1_ref[...], 0.0)
        r2 = jnp.maximum(
            jnp.dot(r1, w2_ref[...], preferred_element_type=jnp.float32)
            + b2_ref[...], 0.0)
        r3 = jnp.sum(r2 * w3_ref[...], axis=1, keepdims=True) + b3_ref[...]
        out_ref[...] = r3

    return pl.pallas_call(
        body,
        grid=(p_pad // BP,),
        in_specs=[
            pl.BlockSpec((BP, _LANES), lambda i: (i, 0)),
            pl.BlockSpec((_LANES, ru), lambda i: (0, 0)),
            pl.BlockSpec((1, ru), lambda i: (0, 0)),
            pl.BlockSpec((ru, ru), lambda i: (0, 0)),
            pl.BlockSpec((1, ru), lambda i: (0, 0)),
            pl.BlockSpec((1, ru), lambda i: (0, 0)),
            pl.BlockSpec((1, 1), lambda i: (0, 0)),
        ],
        out_specs=pl.BlockSpec((BP, 1), lambda i: (i, 0)),
        out_shape=jax.ShapeDtypeStruct((p_pad, 1), jnp.float32),
    )(h, w1_t, b1, w2_t, b2, w3, b3)


def kernel(traffic, packets, time_dist_params, capacity, link_to_path,
           path_to_link, path_ids, sequence_path, sequence_links, n_links,
           n_paths, gru_wih, gru_whh, gru_bih, gru_bhh, cell_wih, cell_whh,
           cell_bih, cell_bhh, W1, b1, W2, b2, W3, b3):
    P = traffic.shape[0]
    L = sequence_path.shape[0] // P
    NL = capacity.shape[0]
    Hl = cell_whh.shape[1]
    Hp = gru_whh.shape[1]
    RU = W1.shape[0]
    Dt = time_dist_params.shape[1]

    p_pad = _round_up(P, 1024)
    nl_pad = _round_up(NL, 512)

    idx2 = link_to_path.reshape(P, L).astype(jnp.int32)
    idx_t = jnp.full((L, p_pad), NL, jnp.int32).at[:, :P].set(idx2.T)
    idx_gather = idx_t.reshape(32, -1, 128)

    link0 = jnp.zeros((nl_pad, Hl), jnp.float32).at[:NL, 0].set(capacity)
    path0 = (jnp.zeros((p_pad, Hl), jnp.float32)
             .at[:P, 0].set(traffic)
             .at[:P, 1].set(packets)
             .at[:P, 2:2 + Dt].set(time_dist_params))

    wih_t = gru_wih.T
    whh_t = gru_whh.T
    bih = gru_bih[None, :]
    bhh = gru_bhh[None, :]
    cwih_t = cell_wih.T
    cwhh_t = cell_whh.T
    cbih = cell_bih[None, :]
    cbhh = cell_bhh[None, :]
    w1_t = W1.T
    w2_t = W2.T
    b1r = b1[None, :]
    b2r = b2[None, :]
    w3r = W3
    b3r = b3[None, :]

    zeros_nl = jnp.zeros((nl_pad, _LANES), jnp.float32)

    link_state = link0
    path_state = path0
    n_rows = L * p_pad
    for _ in range(T_STEPS):
        x = _sc_gather(link_state, idx_gather, n_rows)
        x3 = x.reshape(L, p_pad, _LANES)
        path_state = _path_gru(x3, path_state, wih_t, whh_t, bih, bhh,
                               L, p_pad, Hp)
        parts = _sc_scatter(path_state, idx_gather, zeros_nl,
                            nl_pad, p_pad, L)
        link_state = _link_update(parts[0], parts[1], link_state, cwih_t,
                                  cwhh_t, cbih, cbhh, nl_pad, Hl)

    r = _readout(path_state, w1_t, b1r, w2_t, b2r, w3r, b3r, p_pad, RU)
    return r[:P, 0][None, :]


# speedup vs baseline: 1.0180x; 1.0180x over previous
import functools

import jax
import jax.numpy as jnp
from jax import lax
from jax.experimental import pallas as pl
from jax.experimental.pallas import tpu as pltpu
from jax.experimental.pallas import tpu_sc as plsc

T_STEPS = 4
_LANES = 128


def _round_up(x, m):
    return ((x + m - 1) // m) * m


def _tanh(x):
    x = jnp.clip(x, -7.99881172180175781, 7.99881172180175781)
    x2 = x * x
    p = x2 * -2.76076847742355e-16 + 2.00018790482477e-13
    p = p * x2 + -8.60467152213735e-11
    p = p * x2 + 5.12229709037114e-08
    p = p * x2 + 1.48572235717979e-05
    p = p * x2 + 6.37261928875436e-04
    p = p * x2 + 4.89352455891786e-03
    q = x2 * 1.19825839466702e-06 + 1.18534705686654e-04
    q = q * x2 + 2.26843463243900e-03
    q = q * x2 + 4.89352518554385e-03
    inv = 1.0 / q
    inv = inv * (2.0 - q * inv)
    return x * p * inv


def _sigmoid(x):
    return 0.5 + 0.5 * _tanh(0.5 * x)


def _sc_gather(table, idx3d, n_rows):
    info = plsc.get_sparse_core_info()
    nw = info.num_cores * info.num_subcores
    per = n_rows // nw
    n128 = per // 128
    sub = 1
    chunks = n128 // sub
    nbuf = 3
    mesh = plsc.VectorSubcoreMesh(core_axis_name="c", subcore_axis_name="s")

    n_table = table.shape[0]
    rps = n_table // info.num_subcores

    @functools.partial(
        pl.kernel,
        mesh=mesh,
        out_type=jax.ShapeDtypeStruct((n_rows, _LANES), jnp.float32),
        scratch_types=[
            pltpu.VMEM((n128, 128), jnp.int32),
            pltpu.VMEM((128, _LANES), jnp.float32),
            pltpu.VMEM((128, _LANES), jnp.float32),
            pltpu.VMEM((128, _LANES), jnp.float32),
            pltpu.VMEM_SHARED((n_table, _LANES), jnp.float32),
            pltpu.SemaphoreType.DMA,
            pltpu.SemaphoreType.DMA,
            pltpu.SemaphoreType.DMA,
        ],
    )
    def k(table_hbm, idx_hbm, out_hbm, idx_v, b0, b1, b2, table_sh, s0, s1, s2):
        bufs = (b0, b1, b2)
        sems = (s0, s1, s2)
        cid = lax.axis_index("c")
        sid = lax.axis_index("s")
        wid = sid * info.num_cores + cid
        pltpu.sync_copy(table_hbm.at[pl.ds(sid * rps, rps)],
                        table_sh.at[pl.ds(sid * rps, rps)])
        pltpu.sync_copy(idx_hbm.at[wid], idx_v)
        plsc.subcore_barrier()
        base = wid * per

        def start(j):
            return pltpu.async_copy(
                table_sh.at[idx_v.at[j]],
                bufs[j % nbuf], sems[j % nbuf])

        handles = {}
        for j in range(min(nbuf, chunks)):
            handles[j] = start(j)
        for j in range(chunks):
            handles.pop(j).wait()
            pltpu.sync_copy(bufs[j % nbuf],
                            out_hbm.at[pl.ds(base + j * 128, 128)])
            if j + nbuf < chunks:
                handles[j + nbuf] = start(j + nbuf)

    return k(table, idx3d)


def _sc_scatter(h, idx3d, zeros_nl, nl_pad, p_pad, seq_len):
    info = plsc.get_sparse_core_info()
    nc, ns = info.num_cores, info.num_subcores
    nw = nc * ns
    n_rows = seq_len * p_pad
    per = n_rows // nw
    chunks = per // 128
    half = p_pad // nc
    rps = nl_pad // ns
    mesh = plsc.VectorSubcoreMesh(core_axis_name="c", subcore_axis_name="s")

    @functools.partial(
        pl.kernel,
        mesh=mesh,
        out_type=jax.ShapeDtypeStruct((nc, nl_pad, _LANES), jnp.float32),
        scratch_types=[
            pltpu.VMEM((chunks, 128), jnp.int32),
            pltpu.VMEM((128, _LANES), jnp.float32),
            pltpu.VMEM((128, _LANES), jnp.float32),
            pltpu.VMEM_SHARED((nl_pad, _LANES), jnp.float32),
            pltpu.SemaphoreType.DMA,
            pltpu.SemaphoreType.DMA,
        ],
    )
    def k(h_hbm, idx_hbm, z_hbm, out_hbm, idx_v, r0, r1, accum, s0, s1):
        bufs = (r0, r1)
        sems = (s0, s1)
        cid = lax.axis_index("c")
        sid = lax.axis_index("s")
        wid = sid * nc + cid
        pltpu.sync_copy(z_hbm.at[pl.ds(sid * rps, rps)],
                        accum.at[pl.ds(sid * rps, rps)])
        pltpu.sync_copy(idx_hbm.at[wid], idx_v)
        plsc.subcore_barrier()
        p0 = cid * half

        def start(j):
            return pltpu.async_copy(
                h_hbm.at[pl.ds(p0 + j * 128, 128)], bufs[j % 2], sems[j % 2])

        handles = {}
        for j in range(min(2, chunks)):
            handles[j] = start(j)
        for j in range(chunks):
            handles.pop(j).wait()
            pltpu.sync_copy(bufs[j % 2], accum.at[idx_v.at[j]], add=True)
            if j + 2 < chunks:
                handles[j + 2] = start(j + 2)
        plsc.subcore_barrier()
        pltpu.sync_copy(accum.at[pl.ds(sid * rps, rps)],
                        out_hbm.at[cid, pl.ds(sid * rps, rps)])

    return k(h, idx3d, zeros_nl)


def _path_gru(x3, h, wih_t, whh_t, bih, bhh, seq_len, p_pad, hp):
    BP = 1024
    g3 = 3 * hp

    def body(x_ref, h_ref, wih_ref, whh_ref, bih_ref, bhh_ref, out_ref):
        lens = jnp.zeros((BP,), jnp.int32)
        for t in range(seq_len):
            lens = lens + jnp.any(x_ref[t] != 0.0, axis=1).astype(jnp.int32)
        h_cur = h_ref[...]
        wih = wih_ref[...]
        whh = whh_ref[...]
        bih = bih_ref[...]
        bhh = bhh_ref[...]
        for t in range(seq_len):
            xt = x_ref[t]
            gi = jnp.dot(xt, wih, preferred_element_type=jnp.float32) + bih
            gh = jnp.dot(h_cur, whh, preferred_element_type=jnp.float32) + bhh
            r = jax.nn.sigmoid(gi[:, :hp] + gh[:, :hp])
            z = jax.nn.sigmoid(gi[:, hp:2 * hp] + gh[:, hp:2 * hp])
            n = jnp.tanh(gi[:, 2 * hp:] + r * gh[:, 2 * hp:])
            hn = (1.0 - z) * n + z * h_cur
            h_cur = jnp.where((t < lens)[:, None], hn, h_cur)
        out_ref[...] = h_cur

    return pl.pallas_call(
        body,
        grid=(p_pad // BP,),
        in_specs=[
            pl.BlockSpec((seq_len, BP, _LANES), lambda i: (0, i, 0)),
            pl.BlockSpec((BP, _LANES), lambda i: (i, 0)),
            pl.BlockSpec((_LANES, g3), lambda i: (0, 0)),
            pl.BlockSpec((_LANES, g3), lambda i: (0, 0)),
            pl.BlockSpec((1, g3), lambda i: (0, 0)),
            pl.BlockSpec((1, g3), lambda i: (0, 0)),
        ],
        out_specs=pl.BlockSpec((BP, _LANES), lambda i: (i, 0)),
        out_shape=jax.ShapeDtypeStruct((p_pad, _LANES), jnp.float32),
    )(x3, h, wih_t, whh_t, bih, bhh)


def _link_update(part0, part1, link, cwih_t, cwhh_t, cbih, cbhh, nl_pad, hl):
    BL = 512
    g3 = 3 * hl

    def body(p0_ref, p1_ref, ls_ref, wih_ref, whh_ref, bih_ref, bhh_ref, out_ref):
        ps = p0_ref[...] + p1_ref[...]
        ls = ls_ref[...]
        gi = jnp.dot(ps, wih_ref[...], preferred_element_type=jnp.float32) + bih_ref[...]
        gh = jnp.dot(ls, whh_ref[...], preferred_element_type=jnp.float32) + bhh_ref[...]
        r = jax.nn.sigmoid(gi[:, :hl] + gh[:, :hl])
        z = jax.nn.sigmoid(gi[:, hl:2 * hl] + gh[:, hl:2 * hl])
        n = jnp.tanh(gi[:, 2 * hl:] + r * gh[:, 2 * hl:])
        out_ref[...] = (1.0 - z) * n + z * ls

    return pl.pallas_call(
        body,
        grid=(nl_pad // BL,),
        in_specs=[
            pl.BlockSpec((BL, _LANES), lambda i: (i, 0)),
            pl.BlockSpec((BL, _LANES), lambda i: (i, 0)),
            pl.BlockSpec((BL, _LANES), lambda i: (i, 0)),
            pl.BlockSpec((_LANES, g3), lambda i: (0, 0)),
            pl.BlockSpec((_LANES, g3), lambda i: (0, 0)),
            pl.BlockSpec((1, g3), lambda i: (0, 0)),
            pl.BlockSpec((1, g3), lambda i: (0, 0)),
        ],
        out_specs=pl.BlockSpec((BL, _LANES), lambda i: (i, 0)),
        out_shape=jax.ShapeDtypeStruct((nl_pad, _LANES), jnp.float32),
    )(part0, part1, link, cwih_t, cwhh_t, cbih, cbhh)


def _readout(h, w1_t, b1, w2_t, b2, w3, b3, p_pad, ru):
    BP = 1024

    def body(h_ref, w1_ref, b1_ref, w2_ref, b2_ref, w3_ref, b3_ref, out_ref):
        r1 = jnp.maximum(
            jnp.dot(h_ref[...], w1_ref[...], preferred_element_type=jnp.float32)
            + b1_ref[...], 0.0)
        r2 = jnp.maximum(
            jnp.dot(r1, w2_ref[...], preferred_element_type=jnp.float32)
            + b2_ref[...], 0.0)
        r3 = jnp.sum(r2 * w3_ref[...], axis=1, keepdims=True) + b3_ref[...]
        out_ref[...] = r3

    return pl.pallas_call(
        body,
        grid=(p_pad // BP,),
        in_specs=[
            pl.BlockSpec((BP, _LANES), lambda i: (i, 0)),
            pl.BlockSpec((_LANES, ru), lambda i: (0, 0)),
            pl.BlockSpec((1, ru), lambda i: (0, 0)),
            pl.BlockSpec((ru, ru), lambda i: (0, 0)),
            pl.BlockSpec((1, ru), lambda i: (0, 0)),
            pl.BlockSpec((1, ru), lambda i: (0, 0)),
            pl.BlockSpec((1, 1), lambda i: (0, 0)),
        ],
        out_specs=pl.BlockSpec((BP, 1), lambda i: (i, 0)),
        out_shape=jax.ShapeDtypeStruct((p_pad, 1), jnp.float32),
    )(h, w1_t, b1, w2_t, b2, w3, b3)


def kernel(traffic, packets, time_dist_params, capacity, link_to_path,
           path_to_link, path_ids, sequence_path, sequence_links, n_links,
           n_paths, gru_wih, gru_whh, gru_bih, gru_bhh, cell_wih, cell_whh,
           cell_bih, cell_bhh, W1, b1, W2, b2, W3, b3):
    P = traffic.shape[0]
    L = sequence_path.shape[0] // P
    NL = capacity.shape[0]
    Hl = cell_whh.shape[1]
    Hp = gru_whh.shape[1]
    RU = W1.shape[0]
    Dt = time_dist_params.shape[1]

    p_pad = _round_up(P, 1024)
    nl_pad = _round_up(NL, 512)

    idx2 = link_to_path.reshape(P, L).astype(jnp.int32)
    idx_t = jnp.full((L, p_pad), NL, jnp.int32).at[:, :P].set(idx2.T)
    idx_gather = idx_t.reshape(32, -1, 128)

    link0 = jnp.zeros((nl_pad, Hl), jnp.float32).at[:NL, 0].set(capacity)
    path0 = (jnp.zeros((p_pad, Hl), jnp.float32)
             .at[:P, 0].set(traffic)
             .at[:P, 1].set(packets)
             .at[:P, 2:2 + Dt].set(time_dist_params))

    wih_t = gru_wih.T
    whh_t = gru_whh.T
    bih = gru_bih[None, :]
    bhh = gru_bhh[None, :]
    cwih_t = cell_wih.T
    cwhh_t = cell_whh.T
    cbih = cell_bih[None, :]
    cbhh = cell_bhh[None, :]
    w1_t = W1.T
    w2_t = W2.T
    b1r = b1[None, :]
    b2r = b2[None, :]
    w3r = W3
    b3r = b3[None, :]

    zeros_nl = jnp.zeros((nl_pad, _LANES), jnp.float32)

    link_state = link0
    path_state = path0
    n_rows = L * p_pad
    for _ in range(T_STEPS):
        x = _sc_gather(link_state, idx_gather, n_rows)
        x3 = x.reshape(L, p_pad, _LANES)
        path_state = _path_gru(x3, path_state, wih_t, whh_t, bih, bhh,
                               L, p_pad, Hp)
        parts = _sc_scatter(path_state, idx_gather, zeros_nl,
                            nl_pad, p_pad, L)
        link_state = _link_update(parts[0], parts[1], link_state, cwih_t,
                                  cwhh_t, cbih, cbhh, nl_pad, Hl)

    r = _readout(path_state, w1_t, b1r, w2_t, b2r, w3r, b3r, p_pad, RU)
    return r[:P, 0][None, :]
